# baseline (speedup 1.0000x reference)
import functools

import jax
import jax.numpy as jnp
import numpy as np
from jax import lax
from jax.experimental import pallas as pl
from jax.experimental.pallas import tpu as pltpu
from jax.experimental.pallas import tpu_sc as plsc

B = 16384
F = 26
D = 32
BF = B * F
CARD = 100000

NC = 2
NS = 16
NW = NC * NS
PER_W = BF // NW

R = 1664
NCHUNK = PER_W // R
NSTREAM = R // 128
GROUPS = R // F


def _body(x_hbm, shared_hbm, offpat_hbm, table_hbm, out_hbm,
          idx_v, rows_v, shared_v, offpat_v, sem):
    wid = lax.axis_index("s") * NC + lax.axis_index("c")
    base = wid * PER_W

    pltpu.sync_copy(shared_hbm, shared_v)
    pltpu.sync_copy(offpat_hbm, offpat_v)

    for c in range(NCHUNK):
        pltpu.sync_copy(x_hbm.at[wid * NCHUNK + c], idx_v)
        for j in range(NSTREAM):
            for k in range(128 // 16):
                sl = pl.ds(k * 16, 16)
                idx_v[j, sl] = idx_v[j, sl] + offpat_v[j, sl]

        copies = []
        for j in range(NSTREAM):
            copies.append(pltpu.async_copy(
                table_hbm.at[idx_v.at[j]],
                rows_v.at[pl.ds(j * 128, 128)],
                sem))
        for cp in copies:
            cp.wait()

        def add_group(g, _):
            r0 = g * F
            for r in range(F):
                for col in (0, 16):
                    sl = pl.ds(col, 16)
                    rows_v[r0 + r, sl] = rows_v[r0 + r, sl] + shared_v[r, sl]
            return _
        lax.fori_loop(0, GROUPS, add_group, 0)

        pltpu.sync_copy(rows_v, out_hbm.at[pl.ds(base + c * R, R)])


def kernel(x, indiv_embed, shared_embed):
    offsets = (np.arange(F, dtype=np.int32) * CARD)
    offpat = jnp.asarray(np.tile(offsets, R // F).reshape(NSTREAM, 128))
    x3d = x.reshape(NW * NCHUNK, NSTREAM, 128)

    mesh = plsc.VectorSubcoreMesh(core_axis_name="c", subcore_axis_name="s")
    run = pl.kernel(
        _body,
        out_type=jax.ShapeDtypeStruct((BF, D), jnp.float32),
        mesh=mesh,
        scratch_types=[
            pltpu.VMEM((NSTREAM, 128), jnp.int32),
            pltpu.VMEM((R, D), jnp.float32),
            pltpu.VMEM((F, D), jnp.float32),
            pltpu.VMEM((NSTREAM, 128), jnp.int32),
            pltpu.SemaphoreType.DMA,
        ],
        compiler_params=pltpu.CompilerParams(use_tc_tiling_on_sc=False),
    )
    out = run(x3d, shared_embed, offpat, indiv_embed)
    return out.reshape(B, F, D)
